# baseline (speedup 1.0000x reference)
import functools

import jax
import jax.numpy as jnp
from jax import lax
from jax.experimental import pallas as pl
from jax.experimental.pallas import tpu as pltpu
from jax.experimental.pallas import tpu_sc as plsc

VOCAB = 1000000
EMB = 64
HID = 64
EPS = 1e-5

PACK = VOCAB // 2
TC_COLS = 32768

SHIFT = (TC_COLS // 2).bit_length() - 1

NC, NS = 2, 16
NW = NC * NS
CHUNK = 256


def _transform_body(t_ref, w_ref, b_ref, g_ref, be_ref, o_ref):
    tT = t_ref[...]
    g = jnp.dot(w_ref[...], tT, preferred_element_type=jnp.float32)
    g = g + b_ref[...]
    cols = lax.broadcasted_iota(jnp.int32, (HID, TC_COLS), 1)
    is_pad = (pl.program_id(0) == 0) & (cols == 0)
    g = jnp.where(is_pad, b_ref[...], g)
    m = jnp.mean(g, axis=0, keepdims=True)
    v = jnp.mean((g - m) ** 2, axis=0, keepdims=True)
    n = (g - m) * lax.rsqrt(v + EPS) * g_ref[...] + be_ref[...]
    g128 = jnp.concatenate([n[:, :TC_COLS // 2], n[:, TC_COLS // 2:]],
                           axis=0)
    o_ref[...] = g128.T


GRID = -(-VOCAB // TC_COLS)
PACK_PAD = GRID * (TC_COLS // 2)


def _transform_table(tableT, W, b_col, g_col, be_col):
    grid = GRID
    return pl.pallas_call(
        _transform_body,
        grid=(grid,),
        in_specs=[
            pl.BlockSpec((EMB, TC_COLS), lambda i: (0, i)),
            pl.BlockSpec((HID, EMB), lambda i: (0, 0)),
            pl.BlockSpec((HID, 1), lambda i: (0, 0)),
            pl.BlockSpec((HID, 1), lambda i: (0, 0)),
            pl.BlockSpec((HID, 1), lambda i: (0, 0)),
        ],
        out_specs=pl.BlockSpec((TC_COLS // 2, 2 * HID), lambda i: (i, 0)),
        out_shape=jax.ShapeDtypeStruct((PACK_PAD, 2 * HID), jnp.float32),
    )(tableT, W, b_col, g_col, be_col)


def _gather_rows(table2, idx_flat, n_tok):
    per_w = n_tok // NW
    n_chunks = per_w // CHUNK
    mesh = plsc.VectorSubcoreMesh(core_axis_name="c", subcore_axis_name="s")

    @functools.partial(
        pl.kernel,
        mesh=mesh,
        compiler_params=pltpu.CompilerParams(use_tc_tiling_on_sc=False),
        out_type=jax.ShapeDtypeStruct((n_tok, 2 * HID), jnp.float32),
        scratch_types=[
            pltpu.VMEM((CHUNK,), jnp.int32),
            pltpu.VMEM((CHUNK,), jnp.int32),
            pltpu.VMEM((CHUNK, HID), jnp.float32),
            pltpu.VMEM((CHUNK, HID), jnp.float32),
            pltpu.SemaphoreType.DMA,
            pltpu.SemaphoreType.DMA,
            pltpu.SemaphoreType.DMA,
            pltpu.SemaphoreType.DMA,
            pltpu.SemaphoreType.DMA,
            pltpu.SemaphoreType.DMA,
        ],
    )
    def k(table_hbm, idx_hbm, out_hbm, idx0, idx1, rows0, rows1,
          si0, si1, sg0, sg1, sw0, sw1):
        wid = lax.axis_index("s") * NC + lax.axis_index("c")
        base = wid * per_w
        idx_v = (idx0, idx1)
        rows_v = (rows0, rows1)
        s_idx = (si0, si1)
        s_g = (sg0, sg1)
        s_wb = (sw0, sw1)

        def idx_start(c, b):
            pltpu.async_copy(idx_hbm.at[pl.ds(base + c * CHUNK, CHUNK)],
                             idx_v[b], s_idx[b])

        def idx_remap(b):
            ref = idx_v[b]

            def rbody(k, carry):
                iv = ref[pl.ds(k * 16, 16)]
                u = ((iv & jnp.int32(-TC_COLS))
                     + ((iv & jnp.int32(TC_COLS // 2 - 1)) << 1)
                     + ((iv >> SHIFT) & jnp.int32(1)))
                ref[pl.ds(k * 16, 16)] = u
                return carry

            lax.fori_loop(0, CHUNK // 16, rbody, 0)

        def wb_start(c, b):
            pltpu.async_copy(
                rows_v[b],
                out_hbm.at[pl.ds(base + c * CHUNK, CHUNK), pl.ds(0, HID)],
                s_wb[b])

        idx_start(0, 0)

        def body(j, carry):
            for b in (0, 1):
                c = 2 * j + b
                pltpu.make_async_copy(
                    idx_hbm.at[pl.ds(0, CHUNK)], idx_v[b], s_idx[b]).wait()
                idx_remap(b)

                @pl.when(c >= 2)
                def _():
                    pltpu.make_async_copy(
                        rows_v[b],
                        out_hbm.at[pl.ds(0, CHUNK), pl.ds(0, HID)],
                        s_wb[b]).wait()

                pltpu.async_copy(table_hbm.at[idx_v[b]], rows_v[b], s_g[b])

                @pl.when(c >= 1)
                def _():
                    pltpu.make_async_copy(
                        table_hbm.at[idx_v[1 - b]], rows_v[1 - b],
                        s_g[1 - b]).wait()
                    wb_start(c - 1, 1 - b)

                @pl.when(c + 1 < n_chunks)
                def _():
                    idx_start(c + 1, 1 - b)
            return carry

        lax.fori_loop(0, n_chunks // 2, body, 0)
        last = n_chunks - 1
        bl = last % 2
        pltpu.make_async_copy(table_hbm.at[idx_v[bl]], rows_v[bl],
                              s_g[bl]).wait()
        wb_start(last, bl)
        pltpu.make_async_copy(
            rows_v[0], out_hbm.at[pl.ds(0, CHUNK), pl.ds(0, HID)],
            s_wb[0]).wait()
        pltpu.make_async_copy(
            rows_v[1], out_hbm.at[pl.ds(0, CHUNK), pl.ds(0, HID)],
            s_wb[1]).wait()

    return k(table2, idx_flat)


L_PER_BLK = 4


def _tpose_body(g_ref, o_ref):
    nb = g_ref.shape[0] // L_PER_BLK
    parts = []
    for k in range(L_PER_BLK):
        gT = g_ref[pl.ds(k * nb, nb), :].T
        parts.append(gT[:HID, :])
    o_ref[...] = jnp.concatenate(parts, axis=0)


def _tpose(gathered, n_l, n_b):
    grid = n_l // L_PER_BLK
    return pl.pallas_call(
        _tpose_body,
        grid=(grid,),
        in_specs=[pl.BlockSpec((L_PER_BLK * n_b, 2 * HID), lambda i: (i, 0))],
        out_specs=pl.BlockSpec((L_PER_BLK * HID, n_b), lambda i: (i, 0)),
        out_shape=jax.ShapeDtypeStruct((n_l * HID, n_b), jnp.float32),
    )(gathered)


def kernel(x, table, W, b, gamma, beta):
    B, L = x.shape
    n_tok = B * L
    tableT = table.T
    table2p = _transform_table(tableT, W, b.reshape(HID, 1),
                               gamma.reshape(HID, 1), beta.reshape(HID, 1))
    table2 = table2p.reshape(2 * PACK_PAD, EMB)
    idx_lmaj = x.T.reshape(n_tok).astype(jnp.int32)
    half = n_tok // 2
    g0 = _gather_rows(table2, idx_lmaj[:half], half)
    g1 = _gather_rows(table2, idx_lmaj[half:], half)
    o0 = _tpose(g0, L // 2, B)
    o1 = _tpose(g1, L // 2, B)
    out2d = jnp.concatenate([o0, o1], axis=0)
    return jnp.transpose(out2d.reshape(L, HID, B), (2, 0, 1))


# speedup vs baseline: 1.2998x; 1.2998x over previous
import functools

import jax
import jax.numpy as jnp
from jax import lax
from jax.experimental import pallas as pl
from jax.experimental.pallas import tpu as pltpu
from jax.experimental.pallas import tpu_sc as plsc

VOCAB = 1000000
EMB = 64
HID = 64
EPS = 1e-5

PACK = VOCAB // 2
TC_COLS = 32768

SHIFT = (TC_COLS // 2).bit_length() - 1

NC, NS = 2, 16
NW = NC * NS
CHUNK = 512


def _transform_body(t_ref, w_ref, b_ref, g_ref, be_ref, o_ref):
    tT = t_ref[...]
    g = jnp.dot(w_ref[...], tT, preferred_element_type=jnp.float32)
    g = g + b_ref[...]
    cols = lax.broadcasted_iota(jnp.int32, (HID, TC_COLS), 1)
    is_pad = (pl.program_id(0) == 0) & (cols == 0)
    g = jnp.where(is_pad, b_ref[...], g)
    m = jnp.mean(g, axis=0, keepdims=True)
    v = jnp.mean((g - m) ** 2, axis=0, keepdims=True)
    n = (g - m) * lax.rsqrt(v + EPS) * g_ref[...] + be_ref[...]
    g128 = jnp.concatenate([n[:, :TC_COLS // 2], n[:, TC_COLS // 2:]],
                           axis=0)
    o_ref[...] = g128.T


GRID = -(-VOCAB // TC_COLS)
PACK_PAD = GRID * (TC_COLS // 2)


def _transform_table(tableT, W, b_col, g_col, be_col):
    grid = GRID
    return pl.pallas_call(
        _transform_body,
        grid=(grid,),
        in_specs=[
            pl.BlockSpec((EMB, TC_COLS), lambda i: (0, i)),
            pl.BlockSpec((HID, EMB), lambda i: (0, 0)),
            pl.BlockSpec((HID, 1), lambda i: (0, 0)),
            pl.BlockSpec((HID, 1), lambda i: (0, 0)),
            pl.BlockSpec((HID, 1), lambda i: (0, 0)),
        ],
        out_specs=pl.BlockSpec((TC_COLS // 2, 2 * HID), lambda i: (i, 0)),
        out_shape=jax.ShapeDtypeStruct((PACK_PAD, 2 * HID), jnp.float32),
    )(tableT, W, b_col, g_col, be_col)


def _gather_rows(table2, idx_flat, n_tok):
    per_w = n_tok // NW
    n_chunks = per_w // CHUNK
    mesh = plsc.VectorSubcoreMesh(core_axis_name="c", subcore_axis_name="s")

    @functools.partial(
        pl.kernel,
        mesh=mesh,
        compiler_params=pltpu.CompilerParams(use_tc_tiling_on_sc=False),
        out_type=jax.ShapeDtypeStruct((n_tok, 2 * HID), jnp.float32),
        scratch_types=[
            pltpu.VMEM((CHUNK,), jnp.int32),
            pltpu.VMEM((CHUNK,), jnp.int32),
            pltpu.VMEM((CHUNK, HID), jnp.float32),
            pltpu.VMEM((CHUNK, HID), jnp.float32),
            pltpu.SemaphoreType.DMA,
            pltpu.SemaphoreType.DMA,
            pltpu.SemaphoreType.DMA,
            pltpu.SemaphoreType.DMA,
            pltpu.SemaphoreType.DMA,
            pltpu.SemaphoreType.DMA,
        ],
    )
    def k(table_hbm, idx_hbm, out_hbm, idx0, idx1, rows0, rows1,
          si0, si1, sg0, sg1, sw0, sw1):
        wid = lax.axis_index("s") * NC + lax.axis_index("c")
        base = wid * per_w
        idx_v = (idx0, idx1)
        rows_v = (rows0, rows1)
        s_idx = (si0, si1)
        s_g = (sg0, sg1)
        s_wb = (sw0, sw1)

        def idx_start(c, b):
            pltpu.async_copy(idx_hbm.at[pl.ds(base + c * CHUNK, CHUNK)],
                             idx_v[b], s_idx[b])

        def idx_remap(b):
            ref = idx_v[b]

            def rbody(k, carry):
                iv = ref[pl.ds(k * 16, 16)]
                u = ((iv & jnp.int32(-TC_COLS))
                     + ((iv & jnp.int32(TC_COLS // 2 - 1)) << 1)
                     + ((iv >> SHIFT) & jnp.int32(1)))
                ref[pl.ds(k * 16, 16)] = u
                return carry

            lax.fori_loop(0, CHUNK // 16, rbody, 0)

        def wb_start(c, b):
            pltpu.async_copy(
                rows_v[b],
                out_hbm.at[pl.ds(base + c * CHUNK, CHUNK), pl.ds(0, HID)],
                s_wb[b])

        idx_start(0, 0)

        def body(j, carry):
            for b in (0, 1):
                c = 2 * j + b
                pltpu.make_async_copy(
                    idx_hbm.at[pl.ds(0, CHUNK)], idx_v[b], s_idx[b]).wait()
                idx_remap(b)

                @pl.when(c >= 2)
                def _():
                    pltpu.make_async_copy(
                        rows_v[b],
                        out_hbm.at[pl.ds(0, CHUNK), pl.ds(0, HID)],
                        s_wb[b]).wait()

                pltpu.async_copy(table_hbm.at[idx_v[b]], rows_v[b], s_g[b])

                @pl.when(c >= 1)
                def _():
                    pltpu.make_async_copy(
                        table_hbm.at[idx_v[1 - b]], rows_v[1 - b],
                        s_g[1 - b]).wait()
                    wb_start(c - 1, 1 - b)

                @pl.when(c + 1 < n_chunks)
                def _():
                    idx_start(c + 1, 1 - b)
            return carry

        lax.fori_loop(0, n_chunks // 2, body, 0)
        last = n_chunks - 1
        bl = last % 2
        pltpu.make_async_copy(table_hbm.at[idx_v[bl]], rows_v[bl],
                              s_g[bl]).wait()
        wb_start(last, bl)
        pltpu.make_async_copy(
            rows_v[0], out_hbm.at[pl.ds(0, CHUNK), pl.ds(0, HID)],
            s_wb[0]).wait()
        pltpu.make_async_copy(
            rows_v[1], out_hbm.at[pl.ds(0, CHUNK), pl.ds(0, HID)],
            s_wb[1]).wait()

    return k(table2, idx_flat)


def kernel(x, table, W, b, gamma, beta):
    B, L = x.shape
    n_tok = B * L
    tableT = table.T
    table2p = _transform_table(tableT, W, b.reshape(HID, 1),
                               gamma.reshape(HID, 1), beta.reshape(HID, 1))
    table2 = table2p.reshape(2 * PACK_PAD, EMB)
    idx_flat = x.reshape(n_tok).astype(jnp.int32)
    out_wide = _gather_rows(table2, idx_flat, n_tok)
    return out_wide[:, :HID].reshape(B, L, HID)
